# Initial kernel scaffold; baseline (speedup 1.0000x reference)
import jax
import jax.numpy as jnp
from jax.experimental import pallas as pl


def kernel(init_features, W_init, b_init, W_agg, W_self, b_gnn, W1, b1, W2, b2, W3, b3, node_type, edge_index, out_idx):
    raise NotImplementedError("write your pallas kernel here")


# speedup vs baseline: 29.7887x; 29.7887x over previous
import functools

import jax
import jax.numpy as jnp
from jax import lax
from jax.experimental import pallas as pl
from jax.experimental.pallas import tpu as pltpu
from jax.experimental.pallas import tpu_sc as plsc

_NC = 2
_NS = 16
_L = 16


def _sc_hist_gather(edge_index, node_type, out_idx):
    n = node_type.shape[0]
    e = edge_index.shape[1]
    edge_flat = edge_index.reshape(2 * e)
    k = out_idx.shape[0]
    nw = _NC * _NS
    assert e % (nw * _L) == 0
    ech = e // nw
    full_chunks = ech // 128
    rem = ech - full_chunks * 128
    assert rem % _L == 0
    rem_vecs = rem // _L
    chunks = full_chunks + (1 if rem else 0)
    epad = chunks * 128
    assert k % _NS == 0
    kp = k // _NS
    assert kp % _L == 0
    cnt_sz = 3 * n
    cntp = -(-cnt_sz // (_NS * _L)) * (_NS * _L)
    zslice = cntp // _NS
    dump = cnt_sz

    mesh = plsc.VectorSubcoreMesh(core_axis_name="c", subcore_axis_name="s")

    @functools.partial(
        pl.kernel,
        out_type=[
            jax.ShapeDtypeStruct((_NC, k * 8), jnp.float32),
            jax.ShapeDtypeStruct((k,), jnp.int32),
        ],
        mesh=mesh,
        compiler_params=pltpu.CompilerParams(needs_layout_passes=False),
        scratch_types=[
            pltpu.VMEM((epad,), jnp.int32),
            pltpu.VMEM((epad,), jnp.int32),
            pltpu.VMEM((n,), jnp.int32),
            pltpu.VMEM((kp,), jnp.int32),
            pltpu.VMEM((chunks, 128), jnp.int32),
            pltpu.VMEM((128,), jnp.float32),
            pltpu.VMEM((kp * 8 // 128, 128), jnp.int32),
            pltpu.VMEM((kp * 8,), jnp.float32),
            pltpu.VMEM((kp,), jnp.int32),
            pltpu.VMEM((zslice,), jnp.float32),
            pltpu.VMEM_SHARED((cntp,), jnp.float32),
        ],
    )
    def hist(edge_hbm, nt_hbm, oi_hbm, cnt_out, ty_out,
             src_v, dst_v, nt_v, oi_v, idx_e, ones_v, idxg, stg, stgt, zb,
             counts_sh):
        cid = lax.axis_index("c")
        sid = lax.axis_index("s")
        wid = cid * _NS + sid

        zeros16f = jnp.zeros((_L,), jnp.float32)
        ones16f = jnp.ones((_L,), jnp.float32)
        zeros16i = jnp.zeros((_L,), jnp.int32)
        dump16 = jnp.full((_L,), dump, jnp.int32)
        iota = lax.iota(jnp.int32, _L)

        def zb_body(i, _):
            zb[pl.ds(i * _L, _L)] = zeros16f
            return 0
        lax.fori_loop(0, zslice // _L, zb_body, 0)
        pltpu.sync_copy(zb, counts_sh.at[pl.ds(sid * zslice, zslice)])

        for u in range(128 // _L):
            ones_v[pl.ds(u * _L, _L)] = ones16f

        pltpu.sync_copy(edge_hbm.at[pl.ds(wid * ech, ech)],
                        src_v.at[pl.ds(0, ech)])
        pltpu.sync_copy(edge_hbm.at[pl.ds(e + wid * ech, ech)],
                        dst_v.at[pl.ds(0, ech)])
        pltpu.sync_copy(nt_hbm, nt_v)
        pltpu.sync_copy(oi_hbm.at[pl.ds(sid * kp, kp)], oi_v)
        for v in range(ech // _L, epad // _L):
            src_v[pl.ds(v * _L, _L)] = zeros16i

        def flat_body(c, _):
            for u in range(128 // _L):
                base = c * 128 + u * _L
                s = src_v[pl.ds(base, _L)]
                d = dst_v[pl.ds(base, _L)]
                t = plsc.load_gather(nt_v, [s])
                idx_e[c, pl.ds(u * _L, _L)] = d * 3 + t
            return 0
        lax.fori_loop(0, full_chunks, flat_body, 0)
        if rem:
            c = full_chunks
            for u in range(128 // _L):
                if u < rem_vecs:
                    base = c * 128 + u * _L
                    s = src_v[pl.ds(base, _L)]
                    d = dst_v[pl.ds(base, _L)]
                    t = plsc.load_gather(nt_v, [s])
                    idx_e[c, pl.ds(u * _L, _L)] = d * 3 + t
                else:
                    idx_e[c, pl.ds(u * _L, _L)] = dump16

        plsc.subcore_barrier()

        def scat_body(c, _):
            pltpu.sync_copy(ones_v, counts_sh.at[idx_e.at[c]], add=True)
            return 0
        lax.fori_loop(0, chunks, scat_body, 0)

        plsc.subcore_barrier()

        for v in range(kp // _L):
            o = plsc.load_gather(oi_v, [iota + v * _L])
            t = plsc.load_gather(nt_v, [o])
            stgt[pl.ds(v * _L, _L)] = t
        for v in range(kp * 8 // _L):
            kk = (iota >> 3) + 2 * v
            j = iota & 7
            o = plsc.load_gather(oi_v, [kk])
            gidx = jnp.where(j < 3, o * 3 + j, dump16)
            idxg[v * _L // 128, pl.ds((v * _L) % 128, _L)] = gidx
        for r in range(kp * 8 // 128):
            pltpu.sync_copy(counts_sh.at[idxg.at[r]],
                            stg.at[pl.ds(r * 128, 128)])
        pltpu.sync_copy(stg, cnt_out.at[cid, pl.ds(sid * kp * 8, kp * 8)])

        @pl.when(cid == 0)
        def _():
            pltpu.sync_copy(stgt, ty_out.at[pl.ds(sid * kp, kp)])

    return hist(edge_flat, node_type, out_idx)


def _tc_readout(init_features, W_init, b_init, W_agg, W_self, b_gnn,
                W1, b1, W2, b2, W3, b3, cnt8, ty2):
    k = ty2.shape[0]
    h_dim = W_agg.shape[0]

    def body(if_ref, wi_ref, bi_ref, wa_ref, ws_ref, bg_ref,
             w1_ref, b1_ref, w2_ref, b2_ref, w3_ref, b3_ref,
             cnt_ref, ty_ref, out_ref):
        ce_rows = [if_ref[t:t + 1, :] @ wi_ref[t] + bi_ref[t:t + 1, :]
                   for t in range(3)]
        ce8 = jnp.concatenate(ce_rows + [jnp.zeros((5, h_dim), jnp.float32)],
                              axis=0)
        m8 = ce8 @ wa_ref[...]
        s8 = ce8 @ ws_ref[...] + bg_ref[...]
        cnt = cnt_ref[0] + cnt_ref[1]
        oh = (lax.broadcasted_iota(jnp.int32, (k, 8), 1)
              == ty_ref[...]).astype(jnp.float32)
        h = jnp.maximum(cnt @ m8 + oh @ s8, 0.0)
        h = jnp.maximum(h @ w1_ref[...] + b1_ref[...], 0.0)
        h = jnp.maximum(h @ w2_ref[...] + b2_ref[...], 0.0)
        z = h @ w3_ref[...] + b3_ref[...]
        out_ref[...] = jax.nn.sigmoid(z)

    return pl.pallas_call(
        body,
        out_shape=jax.ShapeDtypeStruct((k, 1), jnp.float32),
    )(init_features, W_init, b_init, W_agg, W_self,
      b_gnn.reshape(1, h_dim), W1, b1.reshape(1, h_dim), W2,
      b2.reshape(1, h_dim), W3, b3.reshape(1, 1), cnt8, ty2)


def kernel(init_features, W_init, b_init, W_agg, W_self, b_gnn,
           W1, b1, W2, b2, W3, b3, node_type, edge_index, out_idx):
    k = out_idx.shape[0]
    nt = node_type.astype(jnp.int32)
    ei = edge_index.astype(jnp.int32)
    oi = out_idx.astype(jnp.int32)
    cnt, ty = _sc_hist_gather(ei, nt, oi)
    out2d = _tc_readout(init_features.astype(jnp.float32),
                        W_init.astype(jnp.float32),
                        b_init.astype(jnp.float32),
                        W_agg.astype(jnp.float32),
                        W_self.astype(jnp.float32),
                        b_gnn.astype(jnp.float32),
                        W1.astype(jnp.float32), b1.astype(jnp.float32),
                        W2.astype(jnp.float32), b2.astype(jnp.float32),
                        W3.astype(jnp.float32), b3.astype(jnp.float32),
                        cnt.reshape(_NC, k, 8), ty.reshape(k, 1))
    return out2d.reshape(k)


# speedup vs baseline: 33.5713x; 1.1270x over previous
import functools

import jax
import jax.numpy as jnp
from jax import lax
from jax.experimental import pallas as pl
from jax.experimental.pallas import tpu as pltpu
from jax.experimental.pallas import tpu_sc as plsc

_NC = 2
_NS = 16
_L = 16


def _sc_hist_gather(edge_index, node_type, out_idx):
    n = node_type.shape[0]
    e = edge_index.shape[1]
    edge_flat = edge_index.reshape(2 * e)
    k = out_idx.shape[0]
    nw = _NC * _NS
    assert e % (nw * _L) == 0
    ech = e // nw
    full_chunks = ech // 128
    rem = ech - full_chunks * 128
    assert rem % _L == 0
    rem_vecs = rem // _L
    chunks = full_chunks + (1 if rem else 0)
    epad = chunks * 128
    assert k % _NS == 0
    kp = k // _NS
    assert kp % _L == 0
    cnt_sz = 3 * n
    cntp = -(-cnt_sz // (_NS * _L)) * (_NS * _L)
    zslice = cntp // _NS
    dump = cnt_sz

    mesh = plsc.VectorSubcoreMesh(core_axis_name="c", subcore_axis_name="s")

    @functools.partial(
        pl.kernel,
        out_type=[
            jax.ShapeDtypeStruct((_NC, k * 8), jnp.float32),
            jax.ShapeDtypeStruct((k,), jnp.int32),
        ],
        mesh=mesh,
        compiler_params=pltpu.CompilerParams(needs_layout_passes=False),
        scratch_types=[
            pltpu.VMEM((epad,), jnp.int32),
            pltpu.VMEM((epad,), jnp.int32),
            pltpu.VMEM((n,), jnp.int32),
            pltpu.VMEM((kp,), jnp.int32),
            pltpu.VMEM((chunks, 128), jnp.int32),
            pltpu.VMEM((128,), jnp.float32),
            pltpu.VMEM((kp * 8 // 128, 128), jnp.int32),
            pltpu.VMEM((kp * 8,), jnp.float32),
            pltpu.VMEM((kp,), jnp.int32),
            pltpu.VMEM((zslice,), jnp.float32),
            pltpu.VMEM_SHARED((cntp,), jnp.float32),
            pltpu.SemaphoreType.DMA,
            pltpu.SemaphoreType.DMA,
            pltpu.SemaphoreType.DMA,
        ],
    )
    def hist(edge_hbm, nt_hbm, oi_hbm, cnt_out, ty_out,
             src_v, dst_v, nt_v, oi_v, idx_e, ones_v, idxg, stg, stgt, zb,
             counts_sh, sem_in, sem_sc, sem_g):
        cid = lax.axis_index("c")
        sid = lax.axis_index("s")
        wid = cid * _NS + sid

        zeros16f = jnp.zeros((_L,), jnp.float32)
        ones16f = jnp.ones((_L,), jnp.float32)
        zeros16i = jnp.zeros((_L,), jnp.int32)
        dump16 = jnp.full((_L,), dump, jnp.int32)
        iota = lax.iota(jnp.int32, _L)

        cp_src = pltpu.async_copy(edge_hbm.at[pl.ds(wid * ech, ech)],
                                  src_v.at[pl.ds(0, ech)], sem_in)
        cp_dst = pltpu.async_copy(edge_hbm.at[pl.ds(e + wid * ech, ech)],
                                  dst_v.at[pl.ds(0, ech)], sem_in)
        cp_nt = pltpu.async_copy(nt_hbm, nt_v, sem_in)
        cp_oi = pltpu.async_copy(oi_hbm.at[pl.ds(sid * kp, kp)], oi_v, sem_in)

        def zb_body(i, _):
            zb[pl.ds(i * _L, _L)] = zeros16f
            return 0
        lax.fori_loop(0, zslice // _L, zb_body, 0)
        pltpu.sync_copy(zb, counts_sh.at[pl.ds(sid * zslice, zslice)])

        for u in range(128 // _L):
            ones_v[pl.ds(u * _L, _L)] = ones16f

        cp_src.wait()
        cp_dst.wait()
        cp_nt.wait()
        cp_oi.wait()
        for v in range(ech // _L, epad // _L):
            src_v[pl.ds(v * _L, _L)] = zeros16i

        plsc.subcore_barrier()

        DEPTH = 8

        def fire(c):
            pltpu.async_copy(ones_v, counts_sh.at[idx_e.at[c]], sem_sc,
                             add=True)

        def drain(c):
            pltpu.make_async_copy(ones_v, counts_sh.at[idx_e.at[c]],
                                  sem_sc).wait()

        def chunk_body(c, _):
            for u in range(128 // _L):
                base = c * 128 + u * _L
                s = src_v[pl.ds(base, _L)]
                d = dst_v[pl.ds(base, _L)]
                t = plsc.load_gather(nt_v, [s])
                idx_e[c, pl.ds(u * _L, _L)] = d * 3 + t
            fire(c)

            @pl.when(c >= DEPTH)
            def _():
                drain(c - DEPTH)
            return 0
        lax.fori_loop(0, full_chunks, chunk_body, 0)
        if rem:
            c = full_chunks
            for u in range(128 // _L):
                if u < rem_vecs:
                    base = c * 128 + u * _L
                    s = src_v[pl.ds(base, _L)]
                    d = dst_v[pl.ds(base, _L)]
                    t = plsc.load_gather(nt_v, [s])
                    idx_e[c, pl.ds(u * _L, _L)] = d * 3 + t
                else:
                    idx_e[c, pl.ds(u * _L, _L)] = dump16
            fire(c)

        def drain_body(c, _):
            drain(c)
            return 0
        lax.fori_loop(max(0, chunks - DEPTH), chunks, drain_body, 0)

        plsc.subcore_barrier()

        for r in range(kp * 8 // 128):
            for m in range(128 // _L):
                v = r * (128 // _L) + m
                kk = (iota >> 3) + 2 * v
                j = iota & 7
                o = plsc.load_gather(oi_v, [kk])
                gidx = jnp.where(j < 3, o * 3 + j, dump16)
                idxg[r, pl.ds(m * _L, _L)] = gidx
            pltpu.async_copy(counts_sh.at[idxg.at[r]],
                             stg.at[pl.ds(r * 128, 128)], sem_g)
        for v in range(kp // _L):
            o = plsc.load_gather(oi_v, [iota + v * _L])
            t = plsc.load_gather(nt_v, [o])
            stgt[pl.ds(v * _L, _L)] = t
        for r in range(kp * 8 // 128):
            pltpu.make_async_copy(counts_sh.at[idxg.at[r]],
                                  stg.at[pl.ds(r * 128, 128)], sem_g).wait()
        pltpu.sync_copy(stg, cnt_out.at[cid, pl.ds(sid * kp * 8, kp * 8)])

        @pl.when(cid == 0)
        def _():
            pltpu.sync_copy(stgt, ty_out.at[pl.ds(sid * kp, kp)])

    return hist(edge_flat, node_type, out_idx)


def _tc_readout(init_features, W_init, b_init, W_agg, W_self, b_gnn,
                W1, b1, W2, b2, W3, b3, cnt8, ty2):
    k = ty2.shape[0]
    h_dim = W_agg.shape[0]

    def body(if_ref, wi_ref, bi_ref, wa_ref, ws_ref, bg_ref,
             w1_ref, b1_ref, w2_ref, b2_ref, w3_ref, b3_ref,
             cnt_ref, ty_ref, out_ref):
        ce_rows = [if_ref[t:t + 1, :] @ wi_ref[t] + bi_ref[t:t + 1, :]
                   for t in range(3)]
        ce8 = jnp.concatenate(ce_rows + [jnp.zeros((5, h_dim), jnp.float32)],
                              axis=0)
        m8 = ce8 @ wa_ref[...]
        s8 = ce8 @ ws_ref[...] + bg_ref[...]
        cnt = cnt_ref[0] + cnt_ref[1]
        oh = (lax.broadcasted_iota(jnp.int32, (k, 8), 1)
              == ty_ref[...]).astype(jnp.float32)
        h = jnp.maximum(cnt @ m8 + oh @ s8, 0.0)
        h = jnp.maximum(h @ w1_ref[...] + b1_ref[...], 0.0)
        h = jnp.maximum(h @ w2_ref[...] + b2_ref[...], 0.0)
        z = h @ w3_ref[...] + b3_ref[...]
        out_ref[...] = jax.nn.sigmoid(z)

    return pl.pallas_call(
        body,
        out_shape=jax.ShapeDtypeStruct((k, 1), jnp.float32),
    )(init_features, W_init, b_init, W_agg, W_self,
      b_gnn.reshape(1, h_dim), W1, b1.reshape(1, h_dim), W2,
      b2.reshape(1, h_dim), W3, b3.reshape(1, 1), cnt8, ty2)


def kernel(init_features, W_init, b_init, W_agg, W_self, b_gnn,
           W1, b1, W2, b2, W3, b3, node_type, edge_index, out_idx):
    k = out_idx.shape[0]
    nt = node_type.astype(jnp.int32)
    ei = edge_index.astype(jnp.int32)
    oi = out_idx.astype(jnp.int32)
    cnt, ty = _sc_hist_gather(ei, nt, oi)
    out2d = _tc_readout(init_features.astype(jnp.float32),
                        W_init.astype(jnp.float32),
                        b_init.astype(jnp.float32),
                        W_agg.astype(jnp.float32),
                        W_self.astype(jnp.float32),
                        b_gnn.astype(jnp.float32),
                        W1.astype(jnp.float32), b1.astype(jnp.float32),
                        W2.astype(jnp.float32), b2.astype(jnp.float32),
                        W3.astype(jnp.float32), b3.astype(jnp.float32),
                        cnt.reshape(_NC, k, 8), ty.reshape(k, 1))
    return out2d.reshape(k)


# speedup vs baseline: 40.1685x; 1.1965x over previous
import functools

import jax
import jax.numpy as jnp
from jax import lax
from jax.experimental import pallas as pl
from jax.experimental.pallas import tpu as pltpu
from jax.experimental.pallas import tpu_sc as plsc

_NC = 2
_NS = 16
_L = 16


def _sc_hist_gather(edge_index, node_type, out_idx):
    n = node_type.shape[0]
    e = edge_index.shape[1]
    k = out_idx.shape[0]
    nw = _NC * _NS
    assert e % 128 == 0
    rows = e // 128
    base_rows = rows // nw
    extra = rows - base_rows * nw
    main_e = base_rows * 128
    chunks = base_rows + (1 if extra else 0)
    epad = chunks * 128
    assert k % _NS == 0
    kp = k // _NS
    assert kp % _L == 0
    cnt_sz = 3 * n
    cntp = -(-cnt_sz // (_NS * _L)) * (_NS * _L)
    zslice = cntp // _NS
    dump = cnt_sz

    mesh = plsc.VectorSubcoreMesh(core_axis_name="c", subcore_axis_name="s")

    @functools.partial(
        pl.kernel,
        out_type=[
            jax.ShapeDtypeStruct((_NC, 8, k), jnp.float32),
            jax.ShapeDtypeStruct((k,), jnp.int32),
        ],
        mesh=mesh,
        compiler_params=pltpu.CompilerParams(needs_layout_passes=False),
        scratch_types=[
            pltpu.VMEM((2, epad), jnp.int32),
            pltpu.VMEM((n,), jnp.int32),
            pltpu.VMEM((kp,), jnp.int32),
            pltpu.VMEM((chunks, 128), jnp.int32),
            pltpu.VMEM((128,), jnp.float32),
            pltpu.VMEM((8, 128), jnp.int32),
            pltpu.VMEM((8, 128), jnp.float32),
            pltpu.VMEM((kp,), jnp.int32),
            pltpu.VMEM((zslice,), jnp.float32),
            pltpu.VMEM_SHARED((cntp,), jnp.float32),
            pltpu.SemaphoreType.DMA,
            pltpu.SemaphoreType.DMA,
            pltpu.SemaphoreType.DMA,
        ],
    )
    def hist(edge_hbm, nt_hbm, oi_hbm, cnt_out, ty_out,
             e2_v, nt_v, oi_v, idx_e, ones_v, idxg, stg, stgt, zb,
             counts_sh, sem_in, sem_sc, sem_g):
        cid = lax.axis_index("c")
        sid = lax.axis_index("s")
        wid = cid * _NS + sid

        zeros16f = jnp.zeros((_L,), jnp.float32)
        ones16f = jnp.ones((_L,), jnp.float32)
        zeros16i = jnp.zeros((_L,), jnp.int32)
        dump16 = jnp.full((_L,), dump, jnp.int32)
        iota = lax.iota(jnp.int32, _L)

        col0 = pl.multiple_of(wid * main_e, 128)
        cp_edge = pltpu.async_copy(edge_hbm.at[:, pl.ds(col0, main_e)],
                                   e2_v.at[:, pl.ds(0, main_e)], sem_in)
        cp_nt = pltpu.async_copy(nt_hbm, nt_v, sem_in)
        cp_oi = pltpu.async_copy(oi_hbm.at[pl.ds(sid * kp, kp)], oi_v, sem_in)

        def zb_body(i, _):
            zb[pl.ds(i * _L, _L)] = zeros16f
            return 0
        lax.fori_loop(0, zslice // _L, zb_body, 0)
        pltpu.sync_copy(zb, counts_sh.at[pl.ds(sid * zslice, zslice)])

        for u in range(128 // _L):
            ones_v[pl.ds(u * _L, _L)] = ones16f

        if extra:
            for r in range(2):
                for u in range(128 // _L):
                    e2_v[r, pl.ds(main_e + u * _L, _L)] = zeros16i

            @pl.when(wid < extra)
            def _():
                tcol = pl.multiple_of((nw * base_rows + wid) * 128, 128)
                pltpu.sync_copy(edge_hbm.at[:, pl.ds(tcol, 128)],
                                e2_v.at[:, pl.ds(main_e, 128)])

        cp_edge.wait()
        cp_nt.wait()
        cp_oi.wait()

        plsc.subcore_barrier()

        DEPTH = 8

        def fire(c):
            pltpu.async_copy(ones_v, counts_sh.at[idx_e.at[c]], sem_sc,
                             add=True)

        def drain(c):
            pltpu.make_async_copy(ones_v, counts_sh.at[idx_e.at[c]],
                                  sem_sc).wait()

        def chunk_body(c, _):
            for u in range(128 // _L):
                base = c * 128 + u * _L
                s = e2_v[0, pl.ds(base, _L)]
                d = e2_v[1, pl.ds(base, _L)]
                t = plsc.load_gather(nt_v, [s])
                idx_e[c, pl.ds(u * _L, _L)] = d * 3 + t
            fire(c)

            @pl.when(c >= DEPTH)
            def _():
                drain(c - DEPTH)
            return 0
        lax.fori_loop(0, base_rows, chunk_body, 0)
        if extra:
            c = base_rows
            m = (wid < extra).astype(jnp.int32)
            im = 1 - m
            for u in range(128 // _L):
                base = c * 128 + u * _L
                s = e2_v[0, pl.ds(base, _L)]
                d = e2_v[1, pl.ds(base, _L)]
                t = plsc.load_gather(nt_v, [s])
                idx_e[c, pl.ds(u * _L, _L)] = (d * 3 + t) * m + dump16 * im
            fire(c)

        def drain_body(c, _):
            drain(c)
            return 0
        lax.fori_loop(max(0, base_rows - DEPTH), chunks, drain_body, 0)

        plsc.subcore_barrier()

        for v in range(kp // _L):
            o = plsc.load_gather(oi_v, [iota + v * _L])
            t = plsc.load_gather(nt_v, [o])
            stgt[pl.ds(v * _L, _L)] = t
            o3 = o * 3
            for j in range(8):
                if j < 3:
                    idxg[j, pl.ds(v * _L, _L)] = o3 + j
                else:
                    idxg[j, pl.ds(v * _L, _L)] = dump16
        for j in range(8):
            pltpu.async_copy(counts_sh.at[idxg.at[j]], stg.at[j], sem_g)
        for j in range(8):
            pltpu.make_async_copy(counts_sh.at[idxg.at[j]], stg.at[j],
                                  sem_g).wait()
        pltpu.sync_copy(stg, cnt_out.at[cid, :, pl.ds(sid * kp, kp)])

        @pl.when(cid == 0)
        def _():
            pltpu.sync_copy(stgt, ty_out.at[pl.ds(sid * kp, kp)])

    return hist(edge_index, node_type, out_idx)


def _tc_readout(init_features, W_init, b_init, W_agg, W_self, b_gnn,
                W1, b1, W2, b2, W3, b3, cnt8, ty):
    k = ty.shape[0]
    h_dim = W_agg.shape[0]

    def body(if_ref, wi_ref, bi_ref, wa_ref, ws_ref, bg_ref,
             w1_ref, b1_ref, w2_ref, b2_ref, w3_ref, b3_ref,
             cnt_ref, ty_ref, out_ref):
        ce_rows = [if_ref[t:t + 1, :] @ wi_ref[t] + bi_ref[t:t + 1, :]
                   for t in range(3)]
        ce8 = jnp.concatenate(ce_rows + [jnp.zeros((5, h_dim), jnp.float32)],
                              axis=0)
        m8 = ce8 @ wa_ref[...]
        s8 = ce8 @ ws_ref[...] + bg_ref[...]
        cnt = cnt_ref[0] + cnt_ref[1]
        oh = (lax.broadcasted_iota(jnp.int32, (8, k), 0)
              == ty_ref[...][None, :]).astype(jnp.float32)
        x = jnp.concatenate([cnt, oh], axis=0)
        w0 = jnp.concatenate([m8, s8], axis=0)
        h = lax.dot_general(x, w0, (((0,), (0,)), ((), ())),
                            precision=lax.Precision.HIGHEST,
                            preferred_element_type=jnp.float32)
        h = jnp.maximum(h, 0.0)
        h = jnp.maximum(h @ w1_ref[...] + b1_ref[...], 0.0)
        h = jnp.maximum(h @ w2_ref[...] + b2_ref[...], 0.0)
        z = h @ w3_ref[...] + b3_ref[...]
        out_ref[...] = jax.nn.sigmoid(z)

    return pl.pallas_call(
        body,
        out_shape=jax.ShapeDtypeStruct((k, 1), jnp.float32),
    )(init_features, W_init, b_init, W_agg, W_self,
      b_gnn.reshape(1, h_dim), W1, b1.reshape(1, h_dim), W2,
      b2.reshape(1, h_dim), W3, b3.reshape(1, 1), cnt8, ty)


def kernel(init_features, W_init, b_init, W_agg, W_self, b_gnn,
           W1, b1, W2, b2, W3, b3, node_type, edge_index, out_idx):
    k = out_idx.shape[0]
    nt = node_type.astype(jnp.int32)
    ei = edge_index.astype(jnp.int32)
    oi = out_idx.astype(jnp.int32)
    cnt, ty = _sc_hist_gather(ei, nt, oi)
    out2d = _tc_readout(init_features.astype(jnp.float32),
                        W_init.astype(jnp.float32),
                        b_init.astype(jnp.float32),
                        W_agg.astype(jnp.float32),
                        W_self.astype(jnp.float32),
                        b_gnn.astype(jnp.float32),
                        W1.astype(jnp.float32), b1.astype(jnp.float32),
                        W2.astype(jnp.float32), b2.astype(jnp.float32),
                        W3.astype(jnp.float32), b3.astype(jnp.float32),
                        cnt, ty)
    return out2d.reshape(k)
